# baseline (speedup 1.0000x reference)
import functools

import jax
import jax.numpy as jnp
from jax import lax
from jax.experimental import pallas as pl
from jax.experimental.pallas import tpu as pltpu
from jax.experimental.pallas import tpu_sc as plsc

E = 8
K = 2
T = 2048
D = 1024
F = 2048
TIL = 128
A_PAD = T * K + E * TIL
NT = A_PAD // TIL
NTE = 64
FCHUNK = 1024
NF = F // FCHUNK
NW = 32
TPW = T // NW
CH = 16
NCH = TPW // CH


def _meta_body(gating_ref, p1_ref, p2_ref, w1s_ref, w2s_ref, te_ref):
    gating = gating_ref[...]
    t, n = gating.shape
    m = jnp.max(gating, axis=1, keepdims=True)
    p = jnp.exp(gating - m)
    rw = p / jnp.sum(p, axis=1, keepdims=True)
    colid = lax.broadcasted_iota(jnp.int32, rw.shape, 1)
    m1 = jnp.max(rw, axis=1, keepdims=True)
    i1 = jnp.min(jnp.where(rw == m1, colid, n), axis=1, keepdims=True)
    is1 = colid == i1
    rw_m = jnp.where(is1, -jnp.inf, rw)
    m2 = jnp.max(rw_m, axis=1, keepdims=True)
    i2 = jnp.min(jnp.where(rw_m == m2, colid, n), axis=1, keepdims=True)
    is2 = colid == i2
    denom = m1 + m2
    v1 = m1 / denom
    v2 = m2 / denom
    selmat = jnp.where(is1 | is2, 1.0, 0.0)

    rid = lax.broadcasted_iota(jnp.int32, (t, t), 0)
    cid = lax.broadcasted_iota(jnp.int32, (t, t), 1)
    stril = jnp.where(rid > cid, 1.0, 0.0).astype(jnp.bfloat16)
    csum = lax.dot_general(stril, selmat.astype(jnp.bfloat16),
                           (((1,), (0,)), ((), ())),
                           preferred_element_type=jnp.float32)

    counts = csum[t - 1:t, :] + selmat[t - 1:t, :]
    padded = jnp.floor((counts + (TIL - 1)) * (1.0 / TIL)).astype(jnp.float32)
    padded = padded * TIL
    r8 = lax.broadcasted_iota(jnp.int32, (E, E), 0)
    c8 = lax.broadcasted_iota(jnp.int32, (E, E), 1)
    sutri8 = jnp.where(r8 < c8, 1.0, 0.0)
    starts = lax.dot_general(padded, sutri8, (((1,), (0,)), ((), ())),
                             preferred_element_type=jnp.float32)

    posmat = starts + csum
    p1 = jnp.sum(jnp.where(is1, posmat, 0.0), axis=1, keepdims=True)
    p2 = jnp.sum(jnp.where(is2 & jnp.logical_not(is1), posmat, 0.0),
                 axis=1, keepdims=True)
    p1_ref[...] = p1.astype(jnp.int32)
    p2_ref[...] = p2.astype(jnp.int32)
    w1s_ref[...] = jnp.broadcast_to(v1, (t, 16))
    w2s_ref[...] = jnp.broadcast_to(v2, (t, 16))

    jrow = lax.broadcasted_iota(jnp.int32, (NTE, E), 0).astype(jnp.float32) * TIL
    ecol = lax.broadcasted_iota(jnp.int32, (NTE, E), 1).astype(jnp.float32)
    inrange = jnp.where((jrow >= starts) & (jrow < starts + padded), 1.0, 0.0)
    te = jnp.sum(ecol * inrange, axis=1, keepdims=True)
    te = te + (E - 1) * (1.0 - jnp.sum(inrange, axis=1, keepdims=True))
    te_ref[...] = te.astype(jnp.int32)


def _meta(gating):
    return pl.pallas_call(
        _meta_body,
        out_shape=(
            jax.ShapeDtypeStruct((T, 1), jnp.int32),
            jax.ShapeDtypeStruct((T, 1), jnp.int32),
            jax.ShapeDtypeStruct((T, 16), jnp.float32),
            jax.ShapeDtypeStruct((T, 16), jnp.float32),
            jax.ShapeDtypeStruct((NTE, 1), jnp.int32),
        ),
    )(gating)


def _gmm_body(te_ref, xb_ref, p1_ref, p2_ref, wg_ref, wu_ref, wd_ref,
              yg_ref, xg_ref, yacc_ref):
    i = pl.program_id(0)
    f = pl.program_id(1)

    @pl.when(f == 0)
    def _():
        base = i * TIL
        rid = lax.broadcasted_iota(jnp.int32, (TIL, T), 0) + base
        hit1 = rid == jnp.broadcast_to(p1_ref[...], (TIL, T))
        hit2 = rid == jnp.broadcast_to(p2_ref[...], (TIL, T))
        pmat = jnp.where(hit1 | hit2, 1.0, 0.0).astype(jnp.bfloat16)
        xg = lax.dot_general(pmat, xb_ref[...], (((1,), (0,)), ((), ())),
                             preferred_element_type=jnp.float32)
        xg_ref[...] = xg.astype(jnp.bfloat16)

    xt = xg_ref[...]
    g = lax.dot_general(xt, wg_ref[0].astype(jnp.bfloat16),
                        (((1,), (1,)), ((), ())),
                        preferred_element_type=jnp.float32)
    u = lax.dot_general(xt, wu_ref[0].astype(jnp.bfloat16),
                        (((1,), (1,)), ((), ())),
                        preferred_element_type=jnp.float32)
    h = (g * jax.nn.sigmoid(g) * u).astype(jnp.bfloat16)
    y = lax.dot_general(h, wd_ref[0].astype(jnp.bfloat16),
                        (((1,), (1,)), ((), ())),
                        preferred_element_type=jnp.float32)

    @pl.when(f == 0)
    def _():
        yacc_ref[...] = y

    @pl.when(f == NF - 1)
    def _():
        yg_ref[...] = yacc_ref[...] + y

    @pl.when((f > 0) & (f < NF - 1))
    def _():
        yacc_ref[...] = yacc_ref[...] + y


def _gmm(xb, p1r, p2r, w13, w2, te):
    grid_spec = pltpu.PrefetchScalarGridSpec(
        num_scalar_prefetch=1,
        grid=(NT, NF),
        in_specs=[
            pl.BlockSpec((T, D), lambda i, f, te: (0, 0)),
            pl.BlockSpec((1, T), lambda i, f, te: (0, 0)),
            pl.BlockSpec((1, T), lambda i, f, te: (0, 0)),
            pl.BlockSpec((1, FCHUNK, D), lambda i, f, te: (te[i], f, 0)),
            pl.BlockSpec((1, FCHUNK, D), lambda i, f, te: (te[i], NF + f, 0)),
            pl.BlockSpec((1, D, FCHUNK), lambda i, f, te: (te[i], 0, f)),
        ],
        out_specs=pl.BlockSpec((TIL, D), lambda i, f, te: (i, 0)),
        scratch_shapes=[
            pltpu.VMEM((TIL, D), jnp.bfloat16),
            pltpu.VMEM((TIL, D), jnp.float32),
        ],
    )
    return pl.pallas_call(
        _gmm_body,
        grid_spec=grid_spec,
        out_shape=jax.ShapeDtypeStruct((A_PAD, D), jnp.float32),
    )(te, xb, p1r, p2r, w13, w13, w2)


def _combine_sc(yg, p1r, p2r, w1r, w2r):
    mesh = plsc.VectorSubcoreMesh(core_axis_name="c", subcore_axis_name="s")

    @functools.partial(
        pl.kernel, mesh=mesh,
        out_type=jax.ShapeDtypeStruct((T, D), jnp.float32),
        scratch_types=[
            pltpu.VMEM((NCH, CH), jnp.int32),
            pltpu.VMEM((NCH, CH), jnp.int32),
            pltpu.VMEM((TPW, 16), jnp.float32),
            pltpu.VMEM((TPW, 16), jnp.float32),
            pltpu.VMEM((CH, D), jnp.float32),
            pltpu.VMEM((CH, D), jnp.float32),
            pltpu.VMEM((CH, D), jnp.float32),
        ],
    )
    def k(yg_hbm, p1_hbm, p2_hbm, w1_hbm, w2_hbm, out_hbm,
          i1_v, i2_v, w1_v, w2_v, r1_v, r2_v, o_v):
        wid = lax.axis_index("s") * 2 + lax.axis_index("c")
        base = wid * TPW
        pltpu.sync_copy(p1_hbm.at[wid], i1_v)
        pltpu.sync_copy(p2_hbm.at[wid], i2_v)
        pltpu.sync_copy(w1_hbm.at[wid], w1_v)
        pltpu.sync_copy(w2_hbm.at[wid], w2_v)

        def chunk(c, carry):
            pltpu.sync_copy(yg_hbm.at[i1_v.at[c]], r1_v)
            pltpu.sync_copy(yg_hbm.at[i2_v.at[c]], r2_v)
            for j in range(CH):
                w1spl = w1_v[c * CH + j]
                w2spl = w2_v[c * CH + j]
                for s in range(D // 16):
                    sl = pl.ds(s * 16, 16)
                    o_v[j, sl] = (w1spl * r1_v[j, sl] + w2spl * r2_v[j, sl])
            pltpu.sync_copy(o_v, out_hbm.at[pl.ds(base + c * CH, CH)])
            return carry

        lax.fori_loop(0, NCH, chunk, 0)

    return k(yg, p1r, p2r, w1r, w2r)


@functools.partial(jax.jit, static_argnames=())
def kernel(x, gating_output, w13, w2):
    p1, p2, w1s, w2s, te = _meta(gating_output)
    xb = x.astype(jnp.bfloat16)
    yg = _gmm(xb, p1.reshape(1, T), p2.reshape(1, T), w13, w2,
              te.reshape(NTE))
    out = _combine_sc(yg,
                      p1.reshape(NW, NCH, CH), p2.reshape(NW, NCH, CH),
                      w1s.reshape(NW, TPW, 16), w2s.reshape(NW, TPW, 16))
    return out


# speedup vs baseline: 1.2520x; 1.2520x over previous
import functools

import jax
import jax.numpy as jnp
from jax import lax
from jax.experimental import pallas as pl
from jax.experimental.pallas import tpu as pltpu
from jax.experimental.pallas import tpu_sc as plsc

E = 8
K = 2
T = 2048
D = 1024
F = 2048
TIL = 128
A_PAD = T * K + E * TIL
NT = A_PAD // TIL
NTE = 64
FCHUNK = 1024
NF = F // FCHUNK
NW = 32
TPW = T // NW
CH = 16
NCH = TPW // CH


def _meta_body(gating_ref, p1_ref, p2_ref, w1s_ref, w2s_ref, te_ref):
    gating = gating_ref[...]
    t, n = gating.shape
    m = jnp.max(gating, axis=1, keepdims=True)
    p = jnp.exp(gating - m)
    rw = p / jnp.sum(p, axis=1, keepdims=True)
    colid = lax.broadcasted_iota(jnp.int32, rw.shape, 1)
    m1 = jnp.max(rw, axis=1, keepdims=True)
    i1 = jnp.min(jnp.where(rw == m1, colid, n), axis=1, keepdims=True)
    is1 = colid == i1
    rw_m = jnp.where(is1, -jnp.inf, rw)
    m2 = jnp.max(rw_m, axis=1, keepdims=True)
    i2 = jnp.min(jnp.where(rw_m == m2, colid, n), axis=1, keepdims=True)
    is2 = colid == i2
    denom = m1 + m2
    v1 = m1 / denom
    v2 = m2 / denom
    selmat = jnp.where(is1 | is2, 1.0, 0.0)

    rid = lax.broadcasted_iota(jnp.int32, (t, t), 0)
    cid = lax.broadcasted_iota(jnp.int32, (t, t), 1)
    stril = jnp.where(rid > cid, 1.0, 0.0).astype(jnp.bfloat16)
    csum = lax.dot_general(stril, selmat.astype(jnp.bfloat16),
                           (((1,), (0,)), ((), ())),
                           preferred_element_type=jnp.float32)

    counts = csum[t - 1:t, :] + selmat[t - 1:t, :]
    padded = jnp.floor((counts + (TIL - 1)) * (1.0 / TIL)).astype(jnp.float32)
    padded = padded * TIL
    r8 = lax.broadcasted_iota(jnp.int32, (E, E), 0)
    c8 = lax.broadcasted_iota(jnp.int32, (E, E), 1)
    sutri8 = jnp.where(r8 < c8, 1.0, 0.0)
    starts = lax.dot_general(padded, sutri8, (((1,), (0,)), ((), ())),
                             preferred_element_type=jnp.float32)

    posmat = starts + csum
    p1 = jnp.sum(jnp.where(is1, posmat, 0.0), axis=1, keepdims=True)
    p2 = jnp.sum(jnp.where(is2 & jnp.logical_not(is1), posmat, 0.0),
                 axis=1, keepdims=True)
    p1_ref[...] = p1.astype(jnp.int32)
    p2_ref[...] = p2.astype(jnp.int32)
    w1s_ref[...] = jnp.broadcast_to(v1, (t, 16))
    w2s_ref[...] = jnp.broadcast_to(v2, (t, 16))

    jrow = lax.broadcasted_iota(jnp.int32, (NTE, E), 0).astype(jnp.float32) * TIL
    ecol = lax.broadcasted_iota(jnp.int32, (NTE, E), 1).astype(jnp.float32)
    inrange = jnp.where((jrow >= starts) & (jrow < starts + padded), 1.0, 0.0)
    te = jnp.sum(ecol * inrange, axis=1, keepdims=True)
    te = te + (E - 1) * (1.0 - jnp.sum(inrange, axis=1, keepdims=True))
    te_ref[...] = te.astype(jnp.int32)


def _meta(gating):
    return pl.pallas_call(
        _meta_body,
        out_shape=(
            jax.ShapeDtypeStruct((T, 1), jnp.int32),
            jax.ShapeDtypeStruct((T, 1), jnp.int32),
            jax.ShapeDtypeStruct((T, 16), jnp.float32),
            jax.ShapeDtypeStruct((T, 16), jnp.float32),
            jax.ShapeDtypeStruct((NTE, 1), jnp.int32),
        ),
    )(gating)


def _dispatch_sc(x, p1, p2):
    mesh = plsc.VectorSubcoreMesh(core_axis_name="c", subcore_axis_name="s")

    @functools.partial(
        pl.kernel, mesh=mesh,
        out_type=jax.ShapeDtypeStruct((A_PAD, D), jnp.float32),
        scratch_types=[
            pltpu.VMEM((TPW, D), jnp.float32),
            pltpu.VMEM((TPW,), jnp.int32),
            pltpu.VMEM((TPW,), jnp.int32),
        ],
    )
    def k(x_hbm, p1_hbm, p2_hbm, xg_hbm, rows_v, i1_v, i2_v):
        wid = lax.axis_index("s") * 2 + lax.axis_index("c")
        base = wid * TPW
        pltpu.sync_copy(x_hbm.at[pl.ds(base, TPW)], rows_v)
        pltpu.sync_copy(p1_hbm.at[pl.ds(base, TPW)], i1_v)
        pltpu.sync_copy(p2_hbm.at[pl.ds(base, TPW)], i2_v)
        pltpu.sync_copy(rows_v, xg_hbm.at[i1_v])
        pltpu.sync_copy(rows_v, xg_hbm.at[i2_v])

    return k(x, p1, p2)


def _gmm_body(te_ref, xg_ref, wg_ref, wu_ref, wd_ref, yg_ref, yacc_ref):
    f = pl.program_id(0)
    i = pl.program_id(1)

    xt = xg_ref[...].astype(jnp.bfloat16)
    g = lax.dot_general(xt, wg_ref[0].astype(jnp.bfloat16),
                        (((1,), (1,)), ((), ())),
                        preferred_element_type=jnp.float32)
    u = lax.dot_general(xt, wu_ref[0].astype(jnp.bfloat16),
                        (((1,), (1,)), ((), ())),
                        preferred_element_type=jnp.float32)
    h = (g * jax.nn.sigmoid(g) * u).astype(jnp.bfloat16)
    y = lax.dot_general(h, wd_ref[0].astype(jnp.bfloat16),
                        (((1,), (1,)), ((), ())),
                        preferred_element_type=jnp.float32)

    @pl.when(f == 0)
    def _():
        yacc_ref[pl.ds(i * TIL, TIL), :] = y

    @pl.when((f > 0) & (f < NF - 1))
    def _():
        yacc_ref[pl.ds(i * TIL, TIL), :] = yacc_ref[pl.ds(i * TIL, TIL), :] + y

    @pl.when(f == NF - 1)
    def _():
        yg_ref[...] = yacc_ref[pl.ds(i * TIL, TIL), :] + y


def _gmm(xg, w13, w2, te):
    grid_spec = pltpu.PrefetchScalarGridSpec(
        num_scalar_prefetch=1,
        grid=(NF, NT),
        in_specs=[
            pl.BlockSpec((TIL, D), lambda f, i, te: (i, 0)),
            pl.BlockSpec((1, FCHUNK, D), lambda f, i, te: (te[i], f, 0)),
            pl.BlockSpec((1, FCHUNK, D), lambda f, i, te: (te[i], NF + f, 0)),
            pl.BlockSpec((1, D, FCHUNK), lambda f, i, te: (te[i], 0, f)),
        ],
        out_specs=pl.BlockSpec((TIL, D), lambda f, i, te: (i, 0)),
        scratch_shapes=[pltpu.VMEM((A_PAD, D), jnp.float32)],
    )
    return pl.pallas_call(
        _gmm_body,
        grid_spec=grid_spec,
        out_shape=jax.ShapeDtypeStruct((A_PAD, D), jnp.float32),
    )(te, xg, w13, w13, w2)


def _combine_sc(yg, p1r, p2r, w1r, w2r):
    mesh = plsc.VectorSubcoreMesh(core_axis_name="c", subcore_axis_name="s")

    @functools.partial(
        pl.kernel, mesh=mesh,
        out_type=jax.ShapeDtypeStruct((T, D), jnp.float32),
        scratch_types=[
            pltpu.VMEM((NCH, CH), jnp.int32),
            pltpu.VMEM((NCH, CH), jnp.int32),
            pltpu.VMEM((TPW, 16), jnp.float32),
            pltpu.VMEM((TPW, 16), jnp.float32),
            pltpu.VMEM((CH, D), jnp.float32),
            pltpu.VMEM((CH, D), jnp.float32),
            pltpu.VMEM((CH, D), jnp.float32),
        ],
    )
    def k(yg_hbm, p1_hbm, p2_hbm, w1_hbm, w2_hbm, out_hbm,
          i1_v, i2_v, w1_v, w2_v, r1_v, r2_v, o_v):
        wid = lax.axis_index("s") * 2 + lax.axis_index("c")
        base = wid * TPW
        pltpu.sync_copy(p1_hbm.at[wid], i1_v)
        pltpu.sync_copy(p2_hbm.at[wid], i2_v)
        pltpu.sync_copy(w1_hbm.at[wid], w1_v)
        pltpu.sync_copy(w2_hbm.at[wid], w2_v)

        def chunk(c, carry):
            pltpu.sync_copy(yg_hbm.at[i1_v.at[c]], r1_v)
            pltpu.sync_copy(yg_hbm.at[i2_v.at[c]], r2_v)
            for j in range(CH):
                w1spl = w1_v[c * CH + j]
                w2spl = w2_v[c * CH + j]
                for s in range(D // 16):
                    sl = pl.ds(s * 16, 16)
                    o_v[j, sl] = (w1spl * r1_v[j, sl] + w2spl * r2_v[j, sl])
            pltpu.sync_copy(o_v, out_hbm.at[pl.ds(base + c * CH, CH)])
            return carry

        lax.fori_loop(0, NCH, chunk, 0)

    return k(yg, p1r, p2r, w1r, w2r)


@functools.partial(jax.jit, static_argnames=())
def kernel(x, gating_output, w13, w2):
    p1, p2, w1s, w2s, te = _meta(gating_output)
    xg = _dispatch_sc(x, p1.reshape(T), p2.reshape(T))
    yg = _gmm(xg, w13, w2, te.reshape(NTE))
    out = _combine_sc(yg,
                      p1.reshape(NW, NCH, CH), p2.reshape(NW, NCH, CH),
                      w1s.reshape(NW, TPW, 16), w2s.reshape(NW, TPW, 16))
    return out


# speedup vs baseline: 1.3418x; 1.0718x over previous
import functools

import jax
import jax.numpy as jnp
from jax import lax
from jax.experimental import pallas as pl
from jax.experimental.pallas import tpu as pltpu
from jax.experimental.pallas import tpu_sc as plsc

E = 8
K = 2
T = 2048
D = 1024
F = 2048
TIL = 128
A_PAD = T * K + E * TIL
NT = A_PAD // TIL
NTE = 64
NW = 32
TPW = T // NW
CH = 16
NCH = TPW // CH


def _meta_body(gating_ref, p1_ref, p2_ref, w1s_ref, w2s_ref, te_ref):
    gating = gating_ref[...]
    t, n = gating.shape
    m = jnp.max(gating, axis=1, keepdims=True)
    p = jnp.exp(gating - m)
    rw = p / jnp.sum(p, axis=1, keepdims=True)
    colid = lax.broadcasted_iota(jnp.int32, rw.shape, 1)
    m1 = jnp.max(rw, axis=1, keepdims=True)
    i1 = jnp.min(jnp.where(rw == m1, colid, n), axis=1, keepdims=True)
    is1 = colid == i1
    rw_m = jnp.where(is1, -jnp.inf, rw)
    m2 = jnp.max(rw_m, axis=1, keepdims=True)
    i2 = jnp.min(jnp.where(rw_m == m2, colid, n), axis=1, keepdims=True)
    is2 = colid == i2
    denom = m1 + m2
    v1 = m1 / denom
    v2 = m2 / denom
    selmat = jnp.where(is1 | is2, 1.0, 0.0)

    rid = lax.broadcasted_iota(jnp.int32, (t, t), 0)
    cid = lax.broadcasted_iota(jnp.int32, (t, t), 1)
    stril = jnp.where(rid > cid, 1.0, 0.0).astype(jnp.bfloat16)
    csum = lax.dot_general(stril, selmat.astype(jnp.bfloat16),
                           (((1,), (0,)), ((), ())),
                           preferred_element_type=jnp.float32)

    counts = csum[t - 1:t, :] + selmat[t - 1:t, :]
    padded = jnp.floor((counts + (TIL - 1)) * (1.0 / TIL)).astype(jnp.float32)
    padded = padded * TIL
    r8 = lax.broadcasted_iota(jnp.int32, (E, E), 0)
    c8 = lax.broadcasted_iota(jnp.int32, (E, E), 1)
    sutri8 = jnp.where(r8 < c8, 1.0, 0.0)
    starts = lax.dot_general(padded, sutri8, (((1,), (0,)), ((), ())),
                             preferred_element_type=jnp.float32)

    posmat = starts + csum
    p1 = jnp.sum(jnp.where(is1, posmat, 0.0), axis=1, keepdims=True)
    p2 = jnp.sum(jnp.where(is2 & jnp.logical_not(is1), posmat, 0.0),
                 axis=1, keepdims=True)
    p1_ref[...] = p1.astype(jnp.int32)
    p2_ref[...] = p2.astype(jnp.int32)
    w1s_ref[...] = jnp.broadcast_to(v1, (t, 16))
    w2s_ref[...] = jnp.broadcast_to(v2, (t, 16))

    jrow = lax.broadcasted_iota(jnp.int32, (NTE, E), 0).astype(jnp.float32) * TIL
    ecol = lax.broadcasted_iota(jnp.int32, (NTE, E), 1).astype(jnp.float32)
    inrange = jnp.where((jrow >= starts) & (jrow < starts + padded), 1.0, 0.0)
    te = jnp.sum(ecol * inrange, axis=1, keepdims=True)
    te = te + (E - 1) * (1.0 - jnp.sum(inrange, axis=1, keepdims=True))
    te_ref[...] = te.astype(jnp.int32)


def _meta(gating):
    return pl.pallas_call(
        _meta_body,
        out_shape=(
            jax.ShapeDtypeStruct((T, 1), jnp.int32),
            jax.ShapeDtypeStruct((T, 1), jnp.int32),
            jax.ShapeDtypeStruct((T, 16), jnp.float32),
            jax.ShapeDtypeStruct((T, 16), jnp.float32),
            jax.ShapeDtypeStruct((NTE, 1), jnp.int32),
        ),
    )(gating)


def _dispatch_sc(x, p1, p2):
    mesh = plsc.VectorSubcoreMesh(core_axis_name="c", subcore_axis_name="s")

    @functools.partial(
        pl.kernel, mesh=mesh,
        out_type=jax.ShapeDtypeStruct((A_PAD, D), jnp.float32),
        scratch_types=[
            pltpu.VMEM((TPW, D), jnp.float32),
            pltpu.VMEM((TPW,), jnp.int32),
            pltpu.VMEM((TPW,), jnp.int32),
        ],
    )
    def k(x_hbm, p1_hbm, p2_hbm, xg_hbm, rows_v, i1_v, i2_v):
        wid = lax.axis_index("s") * 2 + lax.axis_index("c")
        base = wid * TPW
        pltpu.sync_copy(x_hbm.at[pl.ds(base, TPW)], rows_v)
        pltpu.sync_copy(p1_hbm.at[pl.ds(base, TPW)], i1_v)
        pltpu.sync_copy(p2_hbm.at[pl.ds(base, TPW)], i2_v)
        pltpu.sync_copy(rows_v, xg_hbm.at[i1_v])
        pltpu.sync_copy(rows_v, xg_hbm.at[i2_v])

    return k(x, p1, p2)


def _gmm_body(te_ref, xg_ref, wg0_ref, wg1_ref, wu0_ref, wu1_ref,
              wd0_ref, wd1_ref, yg_ref):
    xt = xg_ref[...].astype(jnp.bfloat16)
    y = None
    for wg_ref, wu_ref, wd_ref in ((wg0_ref, wu0_ref, wd0_ref),
                                   (wg1_ref, wu1_ref, wd1_ref)):
        g = lax.dot_general(xt, wg_ref[0].astype(jnp.bfloat16),
                            (((1,), (1,)), ((), ())),
                            preferred_element_type=jnp.float32)
        u = lax.dot_general(xt, wu_ref[0].astype(jnp.bfloat16),
                            (((1,), (1,)), ((), ())),
                            preferred_element_type=jnp.float32)
        h = (g * jax.nn.sigmoid(g) * u).astype(jnp.bfloat16)
        yp = lax.dot_general(h, wd_ref[0].astype(jnp.bfloat16),
                             (((1,), (1,)), ((), ())),
                             preferred_element_type=jnp.float32)
        y = yp if y is None else y + yp
    yg_ref[...] = y


def _gmm(xg, w13, w2, te):
    fh = F // 2
    grid_spec = pltpu.PrefetchScalarGridSpec(
        num_scalar_prefetch=1,
        grid=(NT,),
        in_specs=[
            pl.BlockSpec((TIL, D), lambda i, te: (i, 0)),
            pl.BlockSpec((1, fh, D), lambda i, te: (te[i], 0, 0)),
            pl.BlockSpec((1, fh, D), lambda i, te: (te[i], 1, 0)),
            pl.BlockSpec((1, fh, D), lambda i, te: (te[i], 2, 0)),
            pl.BlockSpec((1, fh, D), lambda i, te: (te[i], 3, 0)),
            pl.BlockSpec((1, D, fh), lambda i, te: (te[i], 0, 0)),
            pl.BlockSpec((1, D, fh), lambda i, te: (te[i], 0, 1)),
        ],
        out_specs=pl.BlockSpec((TIL, D), lambda i, te: (i, 0)),
    )
    return pl.pallas_call(
        _gmm_body,
        grid_spec=grid_spec,
        out_shape=jax.ShapeDtypeStruct((A_PAD, D), jnp.float32),
    )(te, xg, w13, w13, w13, w13, w2, w2)


def _combine_sc(yg, p1r, p2r, w1r, w2r):
    mesh = plsc.VectorSubcoreMesh(core_axis_name="c", subcore_axis_name="s")

    @functools.partial(
        pl.kernel, mesh=mesh,
        out_type=jax.ShapeDtypeStruct((T, D), jnp.float32),
        scratch_types=[
            pltpu.VMEM((NCH, CH), jnp.int32),
            pltpu.VMEM((NCH, CH), jnp.int32),
            pltpu.VMEM((TPW, 16), jnp.float32),
            pltpu.VMEM((TPW, 16), jnp.float32),
            pltpu.VMEM((CH, D), jnp.float32),
            pltpu.VMEM((CH, D), jnp.float32),
            pltpu.VMEM((CH, D), jnp.float32),
        ],
    )
    def k(yg_hbm, p1_hbm, p2_hbm, w1_hbm, w2_hbm, out_hbm,
          i1_v, i2_v, w1_v, w2_v, r1_v, r2_v, o_v):
        wid = lax.axis_index("s") * 2 + lax.axis_index("c")
        base = wid * TPW
        pltpu.sync_copy(p1_hbm.at[wid], i1_v)
        pltpu.sync_copy(p2_hbm.at[wid], i2_v)
        pltpu.sync_copy(w1_hbm.at[wid], w1_v)
        pltpu.sync_copy(w2_hbm.at[wid], w2_v)

        def chunk(c, carry):
            pltpu.sync_copy(yg_hbm.at[i1_v.at[c]], r1_v)
            pltpu.sync_copy(yg_hbm.at[i2_v.at[c]], r2_v)
            for j in range(CH):
                w1spl = w1_v[c * CH + j]
                w2spl = w2_v[c * CH + j]
                for s in range(D // 16):
                    sl = pl.ds(s * 16, 16)
                    o_v[j, sl] = (w1spl * r1_v[j, sl] + w2spl * r2_v[j, sl])
            pltpu.sync_copy(o_v, out_hbm.at[pl.ds(base + c * CH, CH)])
            return carry

        lax.fori_loop(0, NCH, chunk, 0)

    return k(yg, p1r, p2r, w1r, w2r)


@functools.partial(jax.jit, static_argnames=())
def kernel(x, gating_output, w13, w2):
    p1, p2, w1s, w2s, te = _meta(gating_output)
    p1f = p1.reshape(T)
    p2f = p2.reshape(T)
    xg = _dispatch_sc(x, p1f, p2f)
    yg = _gmm(xg, w13, w2, te.reshape(NTE))
    out = _combine_sc(yg,
                      p1.reshape(NW, NCH, CH), p2.reshape(NW, NCH, CH),
                      w1s.reshape(NW, TPW, 16), w2s.reshape(NW, TPW, 16))
    return out


# speedup vs baseline: 1.3582x; 1.0122x over previous
import functools

import jax
import jax.numpy as jnp
from jax import lax
from jax.experimental import pallas as pl
from jax.experimental.pallas import tpu as pltpu
from jax.experimental.pallas import tpu_sc as plsc

E = 8
K = 2
T = 2048
D = 1024
F = 2048
TIL = 128
A_PAD = T * K + E * TIL
NT = A_PAD // TIL
NTE = 64
NW = 32
TPW = T // NW
CH = 16
NCH = TPW // CH


def _meta_body(gating_ref, p1_ref, p2_ref, w1s_ref, w2s_ref, te_ref):
    gating = gating_ref[...]
    t, n = gating.shape
    m = jnp.max(gating, axis=1, keepdims=True)
    p = jnp.exp(gating - m)
    rw = p / jnp.sum(p, axis=1, keepdims=True)
    colid = lax.broadcasted_iota(jnp.int32, rw.shape, 1)
    m1 = jnp.max(rw, axis=1, keepdims=True)
    i1 = jnp.min(jnp.where(rw == m1, colid, n), axis=1, keepdims=True)
    is1 = colid == i1
    rw_m = jnp.where(is1, -jnp.inf, rw)
    m2 = jnp.max(rw_m, axis=1, keepdims=True)
    i2 = jnp.min(jnp.where(rw_m == m2, colid, n), axis=1, keepdims=True)
    is2 = colid == i2
    denom = m1 + m2
    v1 = m1 / denom
    v2 = m2 / denom
    selmat = jnp.where(is1 | is2, 1.0, 0.0)

    rid = lax.broadcasted_iota(jnp.int32, (t, t), 0)
    cid = lax.broadcasted_iota(jnp.int32, (t, t), 1)
    stril = jnp.where(rid > cid, 1.0, 0.0).astype(jnp.bfloat16)
    csum = lax.dot_general(stril, selmat.astype(jnp.bfloat16),
                           (((1,), (0,)), ((), ())),
                           preferred_element_type=jnp.float32)

    counts = csum[t - 1:t, :] + selmat[t - 1:t, :]
    padded = jnp.floor((counts + (TIL - 1)) * (1.0 / TIL)).astype(jnp.float32)
    padded = padded * TIL
    r8 = lax.broadcasted_iota(jnp.int32, (E, E), 0)
    c8 = lax.broadcasted_iota(jnp.int32, (E, E), 1)
    sutri8 = jnp.where(r8 < c8, 1.0, 0.0)
    starts = lax.dot_general(padded, sutri8, (((1,), (0,)), ((), ())),
                             preferred_element_type=jnp.float32)

    posmat = starts + csum
    p1 = jnp.sum(jnp.where(is1, posmat, 0.0), axis=1, keepdims=True)
    p2 = jnp.sum(jnp.where(is2 & jnp.logical_not(is1), posmat, 0.0),
                 axis=1, keepdims=True)
    p1_ref[...] = p1.astype(jnp.int32)
    p2_ref[...] = p2.astype(jnp.int32)
    w1s_ref[...] = jnp.broadcast_to(v1, (t, 16))
    w2s_ref[...] = jnp.broadcast_to(v2, (t, 16))

    jrow = lax.broadcasted_iota(jnp.int32, (NTE, E), 0).astype(jnp.float32) * TIL
    ecol = lax.broadcasted_iota(jnp.int32, (NTE, E), 1).astype(jnp.float32)
    inrange = jnp.where((jrow >= starts) & (jrow < starts + padded), 1.0, 0.0)
    te = jnp.sum(ecol * inrange, axis=1, keepdims=True)
    te = te + (E - 1) * (1.0 - jnp.sum(inrange, axis=1, keepdims=True))
    te_ref[...] = te.astype(jnp.int32)


def _meta(gating):
    return pl.pallas_call(
        _meta_body,
        out_shape=(
            jax.ShapeDtypeStruct((T, 1), jnp.int32),
            jax.ShapeDtypeStruct((T, 1), jnp.int32),
            jax.ShapeDtypeStruct((T, 16), jnp.float32),
            jax.ShapeDtypeStruct((T, 16), jnp.float32),
            jax.ShapeDtypeStruct((NTE, 1), jnp.int32),
        ),
    )(gating)


def _dispatch_sc(x, p1, p2):
    mesh = plsc.VectorSubcoreMesh(core_axis_name="c", subcore_axis_name="s")

    @functools.partial(
        pl.kernel, mesh=mesh,
        out_type=jax.ShapeDtypeStruct((A_PAD, D), jnp.float32),
        scratch_types=[
            pltpu.VMEM((TPW, D), jnp.float32),
            pltpu.VMEM((TPW,), jnp.int32),
            pltpu.VMEM((TPW,), jnp.int32),
        ],
    )
    def k(x_hbm, p1_hbm, p2_hbm, xg_hbm, rows_v, i1_v, i2_v):
        wid = lax.axis_index("s") * 2 + lax.axis_index("c")
        base = wid * TPW
        pltpu.sync_copy(x_hbm.at[pl.ds(base, TPW)], rows_v)
        pltpu.sync_copy(p1_hbm.at[pl.ds(base, TPW)], i1_v)
        pltpu.sync_copy(p2_hbm.at[pl.ds(base, TPW)], i2_v)
        pltpu.sync_copy(rows_v, xg_hbm.at[i1_v])
        pltpu.sync_copy(rows_v, xg_hbm.at[i2_v])

    return k(x, p1, p2)


def _gmm_body(te_ref, xg_ref, wg0_ref, wg1_ref, wu0_ref, wu1_ref,
              wd0_ref, wd1_ref, yg_ref):
    xt = xg_ref[...]
    y = None
    for wg_ref, wu_ref, wd_ref in ((wg0_ref, wu0_ref, wd0_ref),
                                   (wg1_ref, wu1_ref, wd1_ref)):
        g = lax.dot_general(xt, wg_ref[0],
                            (((1,), (1,)), ((), ())),
                            preferred_element_type=jnp.float32)
        u = lax.dot_general(xt, wu_ref[0],
                            (((1,), (1,)), ((), ())),
                            preferred_element_type=jnp.float32)
        h = g * jax.nn.sigmoid(g) * u
        yp = lax.dot_general(h, wd_ref[0],
                             (((1,), (1,)), ((), ())),
                             preferred_element_type=jnp.float32)
        y = yp if y is None else y + yp
    yg_ref[...] = y


def _gmm(xg, w13, w2, te):
    fh = F // 2
    grid_spec = pltpu.PrefetchScalarGridSpec(
        num_scalar_prefetch=1,
        grid=(NT,),
        in_specs=[
            pl.BlockSpec((TIL, D), lambda i, te: (i, 0)),
            pl.BlockSpec((1, fh, D), lambda i, te: (te[i], 0, 0)),
            pl.BlockSpec((1, fh, D), lambda i, te: (te[i], 1, 0)),
            pl.BlockSpec((1, fh, D), lambda i, te: (te[i], 2, 0)),
            pl.BlockSpec((1, fh, D), lambda i, te: (te[i], 3, 0)),
            pl.BlockSpec((1, D, fh), lambda i, te: (te[i], 0, 0)),
            pl.BlockSpec((1, D, fh), lambda i, te: (te[i], 0, 1)),
        ],
        out_specs=pl.BlockSpec((TIL, D), lambda i, te: (i, 0)),
    )
    return pl.pallas_call(
        _gmm_body,
        grid_spec=grid_spec,
        out_shape=jax.ShapeDtypeStruct((A_PAD, D), jnp.float32),
    )(te, xg, w13, w13, w13, w13, w2, w2)


def _combine_sc(yg, p1r, p2r, w1r, w2r):
    mesh = plsc.VectorSubcoreMesh(core_axis_name="c", subcore_axis_name="s")

    @functools.partial(
        pl.kernel, mesh=mesh,
        out_type=jax.ShapeDtypeStruct((T, D), jnp.float32),
        scratch_types=[
            pltpu.VMEM((NCH, CH), jnp.int32),
            pltpu.VMEM((NCH, CH), jnp.int32),
            pltpu.VMEM((TPW, 16), jnp.float32),
            pltpu.VMEM((TPW, 16), jnp.float32),
            pltpu.VMEM((CH, D), jnp.float32),
            pltpu.VMEM((CH, D), jnp.float32),
            pltpu.VMEM((CH, D), jnp.float32),
        ],
    )
    def k(yg_hbm, p1_hbm, p2_hbm, w1_hbm, w2_hbm, out_hbm,
          i1_v, i2_v, w1_v, w2_v, r1_v, r2_v, o_v):
        wid = lax.axis_index("s") * 2 + lax.axis_index("c")
        base = wid * TPW
        pltpu.sync_copy(p1_hbm.at[wid], i1_v)
        pltpu.sync_copy(p2_hbm.at[wid], i2_v)
        pltpu.sync_copy(w1_hbm.at[wid], w1_v)
        pltpu.sync_copy(w2_hbm.at[wid], w2_v)

        def chunk(c, carry):
            pltpu.sync_copy(yg_hbm.at[i1_v.at[c]], r1_v)
            pltpu.sync_copy(yg_hbm.at[i2_v.at[c]], r2_v)
            for j in range(CH):
                w1spl = w1_v[c * CH + j]
                w2spl = w2_v[c * CH + j]
                for s in range(D // 16):
                    sl = pl.ds(s * 16, 16)
                    o_v[j, sl] = (w1spl * r1_v[j, sl] + w2spl * r2_v[j, sl])
            pltpu.sync_copy(o_v, out_hbm.at[pl.ds(base + c * CH, CH)])
            return carry

        lax.fori_loop(0, NCH, chunk, 0)

    return k(yg, p1r, p2r, w1r, w2r)


@functools.partial(jax.jit, static_argnames=())
def kernel(x, gating_output, w13, w2):
    p1, p2, w1s, w2s, te = _meta(gating_output)
    p1f = p1.reshape(T)
    p2f = p2.reshape(T)
    xg = _dispatch_sc(x, p1f, p2f)
    yg = _gmm(xg, w13, w2, te.reshape(NTE))
    out = _combine_sc(yg,
                      p1.reshape(NW, NCH, CH), p2.reshape(NW, NCH, CH),
                      w1s.reshape(NW, TPW, 16), w2s.reshape(NW, TPW, 16))
    return out


# speedup vs baseline: 1.9061x; 1.4034x over previous
import functools

import jax
import jax.numpy as jnp
from jax import lax
from jax.experimental import pallas as pl
from jax.experimental.pallas import tpu as pltpu
from jax.experimental.pallas import tpu_sc as plsc

E = 8
K = 2
T = 2048
D = 1024
F = 2048
TIL = 256
A_PAD = T * K + E * TIL
NT = A_PAD // TIL
NTE = 64
NW = 32
TPW = T // NW
CH = 16
NCH = TPW // CH


def _meta_body(gating_ref, p1_ref, p2_ref, w1s_ref, w2s_ref, te_ref):
    gating = gating_ref[...]
    t, n = gating.shape
    m = jnp.max(gating, axis=1, keepdims=True)
    p = jnp.exp(gating - m)
    rw = p / jnp.sum(p, axis=1, keepdims=True)
    colid = lax.broadcasted_iota(jnp.int32, rw.shape, 1)
    m1 = jnp.max(rw, axis=1, keepdims=True)
    i1 = jnp.min(jnp.where(rw == m1, colid, n), axis=1, keepdims=True)
    is1 = colid == i1
    rw_m = jnp.where(is1, -jnp.inf, rw)
    m2 = jnp.max(rw_m, axis=1, keepdims=True)
    i2 = jnp.min(jnp.where(rw_m == m2, colid, n), axis=1, keepdims=True)
    is2 = colid == i2
    denom = m1 + m2
    v1 = m1 / denom
    v2 = m2 / denom
    selmat = jnp.where(is1 | is2, 1.0, 0.0)

    rid = lax.broadcasted_iota(jnp.int32, (t, t), 0)
    cid = lax.broadcasted_iota(jnp.int32, (t, t), 1)
    stril = jnp.where(rid > cid, 1.0, 0.0).astype(jnp.bfloat16)
    csum = lax.dot_general(stril, selmat.astype(jnp.bfloat16),
                           (((1,), (0,)), ((), ())),
                           preferred_element_type=jnp.float32)

    counts = csum[t - 1:t, :] + selmat[t - 1:t, :]
    padded = jnp.floor((counts + (TIL - 1)) * (1.0 / TIL)).astype(jnp.float32)
    padded = padded * TIL
    r8 = lax.broadcasted_iota(jnp.int32, (E, E), 0)
    c8 = lax.broadcasted_iota(jnp.int32, (E, E), 1)
    sutri8 = jnp.where(r8 < c8, 1.0, 0.0)
    starts = lax.dot_general(padded, sutri8, (((1,), (0,)), ((), ())),
                             preferred_element_type=jnp.float32)

    posmat = starts + csum
    p1 = jnp.sum(jnp.where(is1, posmat, 0.0), axis=1, keepdims=True)
    p2 = jnp.sum(jnp.where(is2 & jnp.logical_not(is1), posmat, 0.0),
                 axis=1, keepdims=True)
    p1_ref[...] = p1.astype(jnp.int32)
    p2_ref[...] = p2.astype(jnp.int32)
    w1s_ref[...] = jnp.broadcast_to(v1, (t, 16))
    w2s_ref[...] = jnp.broadcast_to(v2, (t, 16))

    jrow = lax.broadcasted_iota(jnp.int32, (NTE, E), 0).astype(jnp.float32) * TIL
    ecol = lax.broadcasted_iota(jnp.int32, (NTE, E), 1).astype(jnp.float32)
    inrange = jnp.where((jrow >= starts) & (jrow < starts + padded), 1.0, 0.0)
    te = jnp.sum(ecol * inrange, axis=1, keepdims=True)
    te = te + (E - 1) * (1.0 - jnp.sum(inrange, axis=1, keepdims=True))
    te_ref[...] = te.astype(jnp.int32)


def _meta(gating):
    return pl.pallas_call(
        _meta_body,
        out_shape=(
            jax.ShapeDtypeStruct((T, 1), jnp.int32),
            jax.ShapeDtypeStruct((T, 1), jnp.int32),
            jax.ShapeDtypeStruct((T, 16), jnp.float32),
            jax.ShapeDtypeStruct((T, 16), jnp.float32),
            jax.ShapeDtypeStruct((NTE, 1), jnp.int32),
        ),
    )(gating)


def _dispatch_sc(x, p1, p2):
    mesh = plsc.VectorSubcoreMesh(core_axis_name="c", subcore_axis_name="s")

    @functools.partial(
        pl.kernel, mesh=mesh,
        out_type=jax.ShapeDtypeStruct((A_PAD, D), jnp.float32),
        scratch_types=[
            pltpu.VMEM((TPW, D), jnp.float32),
            pltpu.VMEM((TPW,), jnp.int32),
            pltpu.VMEM((TPW,), jnp.int32),
        ],
    )
    def k(x_hbm, p1_hbm, p2_hbm, xg_hbm, rows_v, i1_v, i2_v):
        wid = lax.axis_index("s") * 2 + lax.axis_index("c")
        base = wid * TPW
        pltpu.sync_copy(x_hbm.at[pl.ds(base, TPW)], rows_v)
        pltpu.sync_copy(p1_hbm.at[pl.ds(base, TPW)], i1_v)
        pltpu.sync_copy(p2_hbm.at[pl.ds(base, TPW)], i2_v)
        pltpu.sync_copy(rows_v, xg_hbm.at[i1_v])
        pltpu.sync_copy(rows_v, xg_hbm.at[i2_v])

    return k(x, p1, p2)


def _gmm_body(te_ref, xg_ref, wg0_ref, wg1_ref, wu0_ref, wu1_ref,
              wd0_ref, wd1_ref, yg_ref):
    xt = xg_ref[...]
    y = None
    for wg_ref, wu_ref, wd_ref in ((wg0_ref, wu0_ref, wd0_ref),
                                   (wg1_ref, wu1_ref, wd1_ref)):
        g = lax.dot_general(xt, wg_ref[0],
                            (((1,), (1,)), ((), ())),
                            preferred_element_type=jnp.float32)
        u = lax.dot_general(xt, wu_ref[0],
                            (((1,), (1,)), ((), ())),
                            preferred_element_type=jnp.float32)
        h = g * jax.nn.sigmoid(g) * u
        yp = lax.dot_general(h, wd_ref[0],
                             (((1,), (1,)), ((), ())),
                             preferred_element_type=jnp.float32)
        y = yp if y is None else y + yp
    yg_ref[...] = y


def _gmm(xg, w13, w2, te):
    fh = F // 2
    grid_spec = pltpu.PrefetchScalarGridSpec(
        num_scalar_prefetch=1,
        grid=(NT,),
        in_specs=[
            pl.BlockSpec((TIL, D), lambda i, te: (i, 0)),
            pl.BlockSpec((1, fh, D), lambda i, te: (te[i], 0, 0)),
            pl.BlockSpec((1, fh, D), lambda i, te: (te[i], 1, 0)),
            pl.BlockSpec((1, fh, D), lambda i, te: (te[i], 2, 0)),
            pl.BlockSpec((1, fh, D), lambda i, te: (te[i], 3, 0)),
            pl.BlockSpec((1, D, fh), lambda i, te: (te[i], 0, 0)),
            pl.BlockSpec((1, D, fh), lambda i, te: (te[i], 0, 1)),
        ],
        out_specs=pl.BlockSpec((TIL, D), lambda i, te: (i, 0)),
    )
    return pl.pallas_call(
        _gmm_body,
        grid_spec=grid_spec,
        out_shape=jax.ShapeDtypeStruct((A_PAD, D), jnp.float32),
    )(te, xg, w13, w13, w13, w13, w2, w2)


def _combine_sc(yg, p1r, p2r, w1r, w2r):
    mesh = plsc.VectorSubcoreMesh(core_axis_name="c", subcore_axis_name="s")

    @functools.partial(
        pl.kernel, mesh=mesh,
        out_type=jax.ShapeDtypeStruct((T, D), jnp.float32),
        scratch_types=[
            pltpu.VMEM((NCH, CH), jnp.int32),
            pltpu.VMEM((NCH, CH), jnp.int32),
            pltpu.VMEM((TPW, 16), jnp.float32),
            pltpu.VMEM((TPW, 16), jnp.float32),
            pltpu.VMEM((CH, D), jnp.float32),
            pltpu.VMEM((CH, D), jnp.float32),
            pltpu.VMEM((CH, D), jnp.float32),
        ],
    )
    def k(yg_hbm, p1_hbm, p2_hbm, w1_hbm, w2_hbm, out_hbm,
          i1_v, i2_v, w1_v, w2_v, r1_v, r2_v, o_v):
        wid = lax.axis_index("s") * 2 + lax.axis_index("c")
        base = wid * TPW
        pltpu.sync_copy(p1_hbm.at[wid], i1_v)
        pltpu.sync_copy(p2_hbm.at[wid], i2_v)
        pltpu.sync_copy(w1_hbm.at[wid], w1_v)
        pltpu.sync_copy(w2_hbm.at[wid], w2_v)

        def chunk(c, carry):
            pltpu.sync_copy(yg_hbm.at[i1_v.at[c]], r1_v)
            pltpu.sync_copy(yg_hbm.at[i2_v.at[c]], r2_v)
            for j in range(CH):
                w1spl = w1_v[c * CH + j]
                w2spl = w2_v[c * CH + j]
                for s in range(D // 16):
                    sl = pl.ds(s * 16, 16)
                    o_v[j, sl] = (w1spl * r1_v[j, sl] + w2spl * r2_v[j, sl])
            pltpu.sync_copy(o_v, out_hbm.at[pl.ds(base + c * CH, CH)])
            return carry

        lax.fori_loop(0, NCH, chunk, 0)

    return k(yg, p1r, p2r, w1r, w2r)


@functools.partial(jax.jit, static_argnames=())
def kernel(x, gating_output, w13, w2):
    p1, p2, w1s, w2s, te = _meta(gating_output)
    p1f = p1.reshape(T)
    p2f = p2.reshape(T)
    xg = _dispatch_sc(x, p1f, p2f)
    yg = _gmm(xg, w13, w2, te.reshape(NTE))
    out = _combine_sc(yg,
                      p1.reshape(NW, NCH, CH), p2.reshape(NW, NCH, CH),
                      w1s.reshape(NW, TPW, 16), w2s.reshape(NW, TPW, 16))
    return out
